# baseline (speedup 1.0000x reference)
import functools

import jax
import jax.numpy as jnp
from jax import lax
from jax.experimental import pallas as pl
from jax.experimental.pallas import tpu as pltpu
from jax.experimental.pallas import tpu_sc as plsc

N = 65536
D_MODEL = 32
D_HALF = 16
D_STATE = 16
NUM_TILES = 8
NUM_STAGES = 16
BATCH = 1024
BN = 2048


def _sc_gather(stage, index, stage_embed, index_embed):
    info = plsc.get_sparse_core_info()
    nc, ns = info.num_cores, info.num_subcores
    nw = nc * ns
    b_per_w = BATCH // nw
    mesh = plsc.VectorSubcoreMesh(core_axis_name="c", subcore_axis_name="s")

    @functools.partial(
        pl.kernel,
        out_type=(
            jax.ShapeDtypeStruct((BATCH, D_HALF), jnp.float32),
            jax.ShapeDtypeStruct((BATCH, D_HALF), jnp.float32),
        ),
        mesh=mesh,
        scratch_types=[
            pltpu.VMEM((b_per_w,), jnp.int32),
            pltpu.VMEM((b_per_w,), jnp.int32),
            pltpu.VMEM((b_per_w, D_HALF), jnp.float32),
            pltpu.VMEM((b_per_w, D_HALF), jnp.float32),
            pltpu.SemaphoreType.DMA,
            pltpu.SemaphoreType.DMA,
        ],
        compiler_params=pltpu.CompilerParams(use_tc_tiling_on_sc=False),
    )
    def k(stage_hbm, index_hbm, sembed_hbm, iembed_hbm, se_out, ie_out,
          sidx_v, iidx_v, srows_v, irows_v, sem_s, sem_i):
        wid = lax.axis_index("s") * nc + lax.axis_index("c")
        base = wid * b_per_w
        pltpu.sync_copy(stage_hbm.at[pl.ds(base, b_per_w)], sidx_v)
        pltpu.sync_copy(index_hbm.at[pl.ds(base, b_per_w)], iidx_v)
        cp_s = pltpu.async_copy(sembed_hbm.at[sidx_v], srows_v, sem_s)
        cp_i = pltpu.async_copy(iembed_hbm.at[iidx_v], irows_v, sem_i)
        cp_s.wait()
        cp_i.wait()
        pltpu.sync_copy(srows_v, se_out.at[pl.ds(base, b_per_w)])
        pltpu.sync_copy(irows_v, ie_out.at[pl.ds(base, b_per_w)])

    return k(stage, index, stage_embed, index_embed)


def _tc_body(se_ref, ie_ref, wr_ref, br_ref, win_ref, wout_ref,
             whead_ref, bhead_ref, logits_ref, tidx_ref, out_s):
    @pl.when(pl.program_id(0) == 0)
    def _prologue():
        se = se_ref[...]
        ie = ie_ref[...]
        x = jnp.concatenate([se, ie], axis=1)
        rl = jnp.dot(x, wr_ref[...], preferred_element_type=jnp.float32) + br_ref[...]
        rl2 = rl * 2.0
        m = jnp.max(rl2, axis=1, keepdims=True)
        e = jnp.exp(rl2 - m)
        probs = e / jnp.sum(e, axis=1, keepdims=True)
        tidx_ref[...] = jnp.argmax(rl, axis=1).astype(jnp.int32)[:, None]
        acc = jnp.zeros((BATCH, D_MODEL), jnp.float32)
        for t in range(NUM_TILES):
            u = jnp.dot(x, win_ref[t], preferred_element_type=jnp.float32)
            y = jnp.dot(u, wout_ref[t], preferred_element_type=jnp.float32)
            acc = acc + probs[:, t:t + 1] * y
        out_s[...] = acc

    logits_ref[...] = jnp.dot(out_s[...], whead_ref[...],
                              preferred_element_type=jnp.float32) + bhead_ref[...]


def _tc_call(se, ie, W_r, b_r, W_in, W_out, W_head, b_head, interpret=False):
    grid = (N // BN,)
    full = lambda shape: pl.BlockSpec(shape, lambda i: (0,) * len(shape))
    return pl.pallas_call(
        _tc_body,
        grid=grid,
        in_specs=[
            full((BATCH, D_HALF)),
            full((BATCH, D_HALF)),
            full((D_MODEL, NUM_TILES)),
            full((1, NUM_TILES)),
            full((NUM_TILES, D_MODEL, D_STATE)),
            full((NUM_TILES, D_STATE, D_MODEL)),
            pl.BlockSpec((D_MODEL, BN), lambda i: (0, i)),
            pl.BlockSpec((1, BN), lambda i: (0, i)),
        ],
        out_specs=[
            pl.BlockSpec((BATCH, BN), lambda i: (0, i)),
            pl.BlockSpec((BATCH, 1), lambda i: (0, 0)),
        ],
        out_shape=[
            jax.ShapeDtypeStruct((BATCH, N), jnp.float32),
            jax.ShapeDtypeStruct((BATCH, 1), jnp.int32),
        ],
        scratch_shapes=[pltpu.VMEM((BATCH, D_MODEL), jnp.float32)],
        interpret=interpret,
    )(se, ie, W_r, b_r.reshape(1, NUM_TILES), W_in, W_out,
      W_head, b_head.reshape(1, N))


def kernel(stage, index, stage_embed, index_embed, W_r, b_r, W_in, A, W_out,
           W_head, b_head):
    del A
    se, ie = _sc_gather(stage.astype(jnp.int32), index.astype(jnp.int32),
                        stage_embed, index_embed)
    logits, tidx = _tc_call(se, ie, W_r, b_r, W_in, W_out, W_head, b_head)
    return logits, tidx.reshape(BATCH)


# speedup vs baseline: 1.0190x; 1.0190x over previous
import functools

import jax
import jax.numpy as jnp
from jax import lax
from jax.experimental import pallas as pl
from jax.experimental.pallas import tpu as pltpu
from jax.experimental.pallas import tpu_sc as plsc

N = 65536
D_MODEL = 32
D_HALF = 16
D_STATE = 16
NUM_TILES = 8
NUM_STAGES = 16
BATCH = 1024
BN = 4096


def _sc_gather(stage, index, stage_embed, index_embed):
    info = plsc.get_sparse_core_info()
    nc, ns = info.num_cores, info.num_subcores
    nw = nc * ns
    b_per_w = BATCH // nw
    mesh = plsc.VectorSubcoreMesh(core_axis_name="c", subcore_axis_name="s")

    @functools.partial(
        pl.kernel,
        out_type=(
            jax.ShapeDtypeStruct((BATCH, D_HALF), jnp.float32),
            jax.ShapeDtypeStruct((BATCH, D_HALF), jnp.float32),
        ),
        mesh=mesh,
        scratch_types=[
            pltpu.VMEM((b_per_w,), jnp.int32),
            pltpu.VMEM((b_per_w,), jnp.int32),
            pltpu.VMEM((b_per_w, D_HALF), jnp.float32),
            pltpu.VMEM((b_per_w, D_HALF), jnp.float32),
            pltpu.SemaphoreType.DMA,
            pltpu.SemaphoreType.DMA,
        ],
        compiler_params=pltpu.CompilerParams(use_tc_tiling_on_sc=False),
    )
    def k(stage_hbm, index_hbm, sembed_hbm, iembed_hbm, se_out, ie_out,
          sidx_v, iidx_v, srows_v, irows_v, sem_s, sem_i):
        wid = lax.axis_index("s") * nc + lax.axis_index("c")
        base = wid * b_per_w
        pltpu.sync_copy(stage_hbm.at[pl.ds(base, b_per_w)], sidx_v)
        pltpu.sync_copy(index_hbm.at[pl.ds(base, b_per_w)], iidx_v)
        cp_s = pltpu.async_copy(sembed_hbm.at[sidx_v], srows_v, sem_s)
        cp_i = pltpu.async_copy(iembed_hbm.at[iidx_v], irows_v, sem_i)
        cp_s.wait()
        cp_i.wait()
        pltpu.sync_copy(srows_v, se_out.at[pl.ds(base, b_per_w)])
        pltpu.sync_copy(irows_v, ie_out.at[pl.ds(base, b_per_w)])

    return k(stage, index, stage_embed, index_embed)


def _tc_body(se_ref, ie_ref, wr_ref, br_ref, win_ref, wout_ref,
             whead_ref, bhead_ref, logits_ref, tidx_ref, out_s):
    @pl.when(pl.program_id(0) == 0)
    def _prologue():
        se = se_ref[...]
        ie = ie_ref[...]
        x = jnp.concatenate([se, ie], axis=1)
        rl = jnp.dot(x, wr_ref[...], preferred_element_type=jnp.float32) + br_ref[...]
        rl2 = rl * 2.0
        m = jnp.max(rl2, axis=1, keepdims=True)
        e = jnp.exp(rl2 - m)
        probs = e / jnp.sum(e, axis=1, keepdims=True)
        tidx_ref[...] = jnp.argmax(rl, axis=1).astype(jnp.int32)[:, None]
        acc = jnp.zeros((BATCH, D_MODEL), jnp.float32)
        for t in range(NUM_TILES):
            u = jnp.dot(x, win_ref[t], preferred_element_type=jnp.float32)
            y = jnp.dot(u, wout_ref[t], preferred_element_type=jnp.float32)
            acc = acc + probs[:, t:t + 1] * y
        out_s[...] = acc

    logits_ref[...] = jnp.dot(out_s[...], whead_ref[...],
                              preferred_element_type=jnp.float32) + bhead_ref[...]


def _tc_call(se, ie, W_r, b_r, W_in, W_out, W_head, b_head, interpret=False):
    grid = (N // BN,)
    full = lambda shape: pl.BlockSpec(shape, lambda i: (0,) * len(shape))
    return pl.pallas_call(
        _tc_body,
        grid=grid,
        in_specs=[
            full((BATCH, D_HALF)),
            full((BATCH, D_HALF)),
            full((D_MODEL, NUM_TILES)),
            full((1, NUM_TILES)),
            full((NUM_TILES, D_MODEL, D_STATE)),
            full((NUM_TILES, D_STATE, D_MODEL)),
            pl.BlockSpec((D_MODEL, BN), lambda i: (0, i)),
            pl.BlockSpec((1, BN), lambda i: (0, i)),
        ],
        out_specs=[
            pl.BlockSpec((BATCH, BN), lambda i: (0, i)),
            pl.BlockSpec((BATCH, 1), lambda i: (0, 0)),
        ],
        out_shape=[
            jax.ShapeDtypeStruct((BATCH, N), jnp.float32),
            jax.ShapeDtypeStruct((BATCH, 1), jnp.int32),
        ],
        scratch_shapes=[pltpu.VMEM((BATCH, D_MODEL), jnp.float32)],
        interpret=interpret,
    )(se, ie, W_r, b_r.reshape(1, NUM_TILES), W_in, W_out,
      W_head, b_head.reshape(1, N))


def kernel(stage, index, stage_embed, index_embed, W_r, b_r, W_in, A, W_out,
           W_head, b_head):
    del A
    se, ie = _sc_gather(stage.astype(jnp.int32), index.astype(jnp.int32),
                        stage_embed, index_embed)
    logits, tidx = _tc_call(se, ie, W_r, b_r, W_in, W_out, W_head, b_head)
    return logits, tidx.reshape(BATCH)
